# baseline (speedup 1.0000x reference)
import functools

import jax
import jax.numpy as jnp
from jax import lax
from jax.experimental import pallas as pl
from jax.experimental.pallas import tpu as pltpu
from jax.experimental.pallas import tpu_sc as plsc


def kernel(x, embedding):
    B, H = x.shape
    V, D = embedding.shape
    N = B * H

    info = plsc.get_sparse_core_info()
    NC, NS, L = info.num_cores, info.num_subcores, info.num_lanes
    NW = NC * NS

    K = 128
    NG = 4
    TL = 2
    NSB = 4
    DT, DI = D // 8, 8
    assert B == NW * K and D == DT * DI
    RBYTES = K * D * 4

    xT = jnp.swapaxes(x, 0, 1).astype(jnp.int32)

    mesh = plsc.VectorSubcoreMesh(core_axis_name="c", subcore_axis_name="s")

    @functools.partial(
        pl.kernel,
        out_type=jax.ShapeDtypeStruct((H, DT, NW, DI, K), jnp.float32),
        mesh=mesh,
        scratch_types=[
            pltpu.VMEM((H, K), jnp.int32),
            pltpu.VMEM((NG, K, D), jnp.float32),
            pltpu.VMEM((NSB, DT, DI, K), jnp.float32),
            pltpu.SemaphoreType.DMA((NG,)),
            pltpu.SemaphoreType.DMA((NSB,)),
        ],
        compiler_params=pltpu.CompilerParams(
            use_tc_tiling_on_sc=False, needs_layout_passes=False
        ),
    )
    def emb_kernel(idx_hbm, table_hbm, out_hbm, idx_v, rows_v, tout_v, gsem, ssem):
        wid = lax.axis_index("s") * NC + lax.axis_index("c")
        pltpu.sync_copy(idx_hbm.at[:, pl.ds(wid * K, K)], idx_v)

        lane = lax.iota(jnp.int32, L)

        def transpose_store(ht):
            bg = jnp.bitwise_and(ht, NG - 1)
            bs = jnp.bitwise_and(ht, NSB - 1)

            @pl.when(ht >= NSB)
            def _():
                pltpu.make_async_copy(
                    tout_v.at[bs], out_hbm.at[0, :, wid], ssem.at[bs]
                ).wait()

            pltpu.make_async_copy(
                table_hbm.at[idx_v.at[ht]], rows_v.at[bg], gsem.at[bg]
            ).wait()

            src = rows_v.at[bg]
            dst = tout_v.at[bs]
            for dt in range(DT):
                for di in range(DI):
                    col = jnp.full((L,), dt * DI + di, jnp.int32)
                    for b0 in range(K // L):
                        vals = plsc.load_gather(src, [lane + (b0 * L), col])
                        dst[dt, di, pl.ds(b0 * L, L)] = vals

            pltpu.async_copy(tout_v.at[bs], out_hbm.at[ht, :, wid], ssem.at[bs])

        def body(h, carry):
            @pl.when(h < H)
            def _():
                bg = jnp.bitwise_and(h, NG - 1)
                pltpu.async_copy(
                    table_hbm.at[idx_v.at[h]], rows_v.at[bg], gsem.at[bg]
                )

            @pl.when(h >= TL)
            def _():
                transpose_store(h - TL)

            return carry

        lax.fori_loop(0, H + TL, body, 0)
        for b in range(NSB):
            pltpu.make_async_copy(
                tout_v.at[b], out_hbm.at[0, :, wid], ssem.at[b]
            ).wait()

    out5 = emb_kernel(xT, embedding)
    return out5.transpose(2, 4, 0, 1, 3).reshape(B, H, D)


# speedup vs baseline: 1.4640x; 1.4640x over previous
import functools

import jax
import jax.numpy as jnp
from jax import lax
from jax.experimental import pallas as pl
from jax.experimental.pallas import tpu as pltpu
from jax.experimental.pallas import tpu_sc as plsc


def kernel(x, embedding):
    B, H = x.shape
    V, D = embedding.shape
    N = B * H

    info = plsc.get_sparse_core_info()
    NC, NS, L = info.num_cores, info.num_subcores, info.num_lanes
    NW = NC * NS

    K = 128
    NG = 4
    TL = 2
    NSB = 4
    DT, DI = D // 8, 8
    assert B == NW * K and D == DT * DI
    RBYTES = K * D * 4

    xT = jnp.swapaxes(x, 0, 1).astype(jnp.int32)

    mesh = plsc.VectorSubcoreMesh(core_axis_name="c", subcore_axis_name="s")

    @functools.partial(
        pl.kernel,
        out_type=jax.ShapeDtypeStruct((H, DT, NW, DI, K), jnp.float32),
        mesh=mesh,
        scratch_types=[
            pltpu.VMEM((H, K), jnp.int32),
            pltpu.VMEM((NG, K, D), jnp.float32),
            pltpu.VMEM((NSB, DT, DI, K), jnp.float32),
            pltpu.SemaphoreType.DMA((NG,)),
            pltpu.SemaphoreType.DMA((NSB,)),
        ],
        compiler_params=pltpu.CompilerParams(
            use_tc_tiling_on_sc=False, needs_layout_passes=False
        ),
    )
    def emb_kernel(idx_hbm, table_hbm, out_hbm, idx_v, rows_v, tout_v, gsem, ssem):
        wid = lax.axis_index("s") * NC + lax.axis_index("c")
        pltpu.sync_copy(idx_hbm.at[:, pl.ds(wid * K, K)], idx_v)

        lane = lax.iota(jnp.int32, L)

        def transpose_store(ht):
            bg = jnp.bitwise_and(ht, NG - 1)
            bs = jnp.bitwise_and(ht, NSB - 1)

            @pl.when(ht >= NSB)
            def _():
                pltpu.make_async_copy(
                    tout_v.at[bs], out_hbm.at[0, :, wid], ssem.at[bs]
                ).wait()

            pltpu.make_async_copy(
                table_hbm.at[idx_v.at[ht]], rows_v.at[bg], gsem.at[bg]
            ).wait()

            src = rows_v.at[bg]
            dst = tout_v.at[bs]

            @plsc.parallel_loop(0, D, unroll=4)
            def _(d):
                col = jax.lax.broadcast(d, (L,))
                dt = jax.lax.shift_right_logical(d, 3)
                di = jnp.bitwise_and(d, DI - 1)
                for b0 in range(K // L):
                    vals = plsc.load_gather(src, [lane + (b0 * L), col])
                    dst[dt, di, pl.ds(b0 * L, L)] = vals

            pltpu.async_copy(tout_v.at[bs], out_hbm.at[ht, :, wid], ssem.at[bs])

        def body(h, carry):
            @pl.when(h < H)
            def _():
                bg = jnp.bitwise_and(h, NG - 1)
                pltpu.async_copy(
                    table_hbm.at[idx_v.at[h]], rows_v.at[bg], gsem.at[bg]
                )

            @pl.when(h >= TL)
            def _():
                transpose_store(h - TL)

            return carry

        lax.fori_loop(0, H + TL, body, 0)
        for b in range(NSB):
            pltpu.make_async_copy(
                tout_v.at[b], out_hbm.at[0, :, wid], ssem.at[b]
            ).wait()

    out5 = emb_kernel(xT, embedding)
    return out5.transpose(2, 4, 0, 1, 3).reshape(B, H, D)


# speedup vs baseline: 2.4297x; 1.6596x over previous
import functools

import jax
import jax.numpy as jnp
from jax import lax
from jax.experimental import pallas as pl
from jax.experimental.pallas import tpu as pltpu
from jax.experimental.pallas import tpu_sc as plsc


def kernel(x, embedding):
    B, H = x.shape
    V, D = embedding.shape
    N = B * H

    info = plsc.get_sparse_core_info()
    NC, NS, L = info.num_cores, info.num_subcores, info.num_lanes
    NW = NC * NS

    K = 128
    NG = 4
    TL = 2
    NSB = 4
    DT, DI = D // 8, 8
    assert B == NW * K and D == DT * DI
    RBYTES = K * D * 4

    xT = jnp.swapaxes(x, 0, 1).astype(jnp.int32)

    mesh = plsc.VectorSubcoreMesh(core_axis_name="c", subcore_axis_name="s")

    @functools.partial(
        pl.kernel,
        out_type=jax.ShapeDtypeStruct((H, DT, NW, DI, K), jnp.float32),
        mesh=mesh,
        scratch_types=[
            pltpu.VMEM((H, K), jnp.int32),
            pltpu.VMEM((NG, K, D), jnp.float32),
            pltpu.VMEM((NSB, DT, DI, K + 1), jnp.float32),
            pltpu.SemaphoreType.DMA((NG,)),
            pltpu.SemaphoreType.DMA((NSB,)),
        ],
        compiler_params=pltpu.CompilerParams(
            use_tc_tiling_on_sc=False, needs_layout_passes=False
        ),
    )
    def emb_kernel(idx_hbm, table_hbm, out_hbm, idx_v, rows_v, tout_v, gsem, ssem):
        wid = lax.axis_index("s") * NC + lax.axis_index("c")
        pltpu.sync_copy(idx_hbm.at[:, pl.ds(wid * K, K)], idx_v)

        lane = lax.iota(jnp.int32, L)
        dtv = [lax.shift_right_logical(lane + d0 * L, 3) for d0 in range(D // L)]
        div = [jnp.bitwise_and(lane + d0 * L, DI - 1) for d0 in range(D // L)]

        def transpose_store(ht):
            bg = jnp.bitwise_and(ht, NG - 1)
            bs = jnp.bitwise_and(ht, NSB - 1)

            @pl.when(ht >= NSB)
            def _():
                pltpu.make_async_copy(
                    tout_v.at[bs, :, :, pl.ds(0, K)],
                    out_hbm.at[0, :, wid],
                    ssem.at[bs],
                ).wait()

            pltpu.make_async_copy(
                table_hbm.at[idx_v.at[ht]], rows_v.at[bg], gsem.at[bg]
            ).wait()

            src = rows_v.at[bg]
            dst = tout_v.at[bs]

            @plsc.parallel_loop(0, K, unroll=2)
            def _(b):
                col = jax.lax.broadcast(b, (L,))
                for d0 in range(D // L):
                    vals = src[b, pl.ds(d0 * L, L)]
                    plsc.store_scatter(dst, [dtv[d0], div[d0], col], vals)

            pltpu.async_copy(
                tout_v.at[bs, :, :, pl.ds(0, K)],
                out_hbm.at[ht, :, wid],
                ssem.at[bs],
            )

        def body(h, carry):
            @pl.when(h < H)
            def _():
                bg = jnp.bitwise_and(h, NG - 1)
                pltpu.async_copy(
                    table_hbm.at[idx_v.at[h]], rows_v.at[bg], gsem.at[bg]
                )

            @pl.when(h >= TL)
            def _():
                transpose_store(h - TL)

            return carry

        lax.fori_loop(0, H + TL, body, 0)
        for b in range(NSB):
            pltpu.make_async_copy(
                tout_v.at[b, :, :, pl.ds(0, K)], out_hbm.at[0, :, wid], ssem.at[b]
            ).wait()

    out5 = emb_kernel(xT, embedding)
    return out5.transpose(2, 4, 0, 1, 3).reshape(B, H, D)
